# baseline (speedup 1.0000x reference)
import functools

import jax
import jax.numpy as jnp
from jax import lax
from jax.experimental import pallas as pl
from jax.experimental.pallas import tpu as pltpu
from jax.experimental.pallas import tpu_sc as plsc

_D = 64
_B = 16384
_NC, _NS, _L = 2, 16, 16
_NW = _NC * _NS
_BPW = _B // _NW
_CH = 128
_NCH = _BPW // _CH

_mesh = plsc.VectorSubcoreMesh(core_axis_name="c", subcore_axis_name="s")


@functools.partial(
    pl.kernel,
    mesh=_mesh,
    out_type=jax.ShapeDtypeStruct((_B,), jnp.float32),
    compiler_params=pltpu.CompilerParams(
        needs_layout_passes=False, use_tc_tiling_on_sc=False),
    scratch_types=[
        pltpu.VMEM((_NCH, _CH), jnp.int32),
        pltpu.VMEM((_NCH, _CH), jnp.int32),
        pltpu.VMEM((_BPW, _D), jnp.float32),
        pltpu.VMEM((_BPW, _D), jnp.float32),
        pltpu.VMEM((_BPW,), jnp.float32),
        pltpu.SemaphoreType.DMA,
    ],
)
def _w2v(center_h, context_h, in_h, oute_h, o_h, cidx, xidx, vbuf, ubuf,
         obuf, sem):
    wid = lax.axis_index("s") * _NC + lax.axis_index("c")
    base = wid * _BPW

    pltpu.sync_copy(center_h.at[pl.ds(wid * _NCH, _NCH)], cidx)
    pltpu.sync_copy(context_h.at[pl.ds(wid * _NCH, _NCH)], xidx)

    copies = []
    for j in range(_NCH):
        copies.append(pltpu.async_copy(
            in_h.at[cidx.at[j]], vbuf.at[pl.ds(j * _CH, _CH)], sem))
        copies.append(pltpu.async_copy(
            oute_h.at[xidx.at[j]], ubuf.at[pl.ds(j * _CH, _CH)], sem))
    for c in copies:
        c.wait()

    iota = lax.iota(jnp.int32, _L)
    last = jnp.full((_L,), _L - 1, jnp.int32)

    def group_body(g, carry):
        outv = jnp.zeros((_L,), jnp.float32)
        for k in range(_L):
            b = g * _L + k
            acc = vbuf[b, pl.ds(0, _L)] * ubuf[b, pl.ds(0, _L)]
            for c in range(1, _D // _L):
                acc = acc + vbuf[b, pl.ds(c * _L, _L)] * ubuf[b, pl.ds(c * _L, _L)]
            tot = jnp.cumsum(acc)
            bcast = tot.at[last].get(mode="promise_in_bounds")
            outv = jnp.where(iota == k, bcast, outv)
        obuf[pl.ds(g * _L, _L)] = outv
        return carry

    lax.fori_loop(0, _BPW // _L, group_body, 0)

    pltpu.sync_copy(obuf, o_h.at[pl.ds(base, _BPW)])


def kernel(center, context, in_embed, out_embed):
    c2 = center.astype(jnp.int32).reshape(_NW * _NCH, _CH)
    x2 = context.astype(jnp.int32).reshape(_NW * _NCH, _CH)
    return _w2v(c2, x2, in_embed, out_embed)


# speedup vs baseline: 1.5787x; 1.5787x over previous
import functools

import jax
import jax.numpy as jnp
from jax import lax
from jax.experimental import pallas as pl
from jax.experimental.pallas import tpu as pltpu
from jax.experimental.pallas import tpu_sc as plsc

_D = 64
_B = 16384
_NC, _NS, _L = 2, 16, 16
_NW = _NC * _NS
_BPW = _B // _NW

_mesh = plsc.VectorSubcoreMesh(core_axis_name="c", subcore_axis_name="s")


@functools.partial(
    pl.kernel,
    mesh=_mesh,
    out_type=jax.ShapeDtypeStruct((_B,), jnp.float32),
    compiler_params=pltpu.CompilerParams(
        needs_layout_passes=False, use_tc_tiling_on_sc=True),
    scratch_types=[
        pltpu.VMEM((_BPW,), jnp.int32),
        pltpu.VMEM((_BPW,), jnp.int32),
        pltpu.VMEM((_BPW // 2, 2 * _D), jnp.float32),
        pltpu.VMEM((_BPW // 2, 2 * _D), jnp.float32),
        pltpu.VMEM((_BPW,), jnp.float32),
        pltpu.SemaphoreType.DMA,
        pltpu.SemaphoreType.DMA,
    ],
)
def _w2v(center_h, context_h, in_h, oute_h, o_h, cidx, xidx, vbuf, ubuf,
         obuf, sem_v, sem_u):
    wid = lax.axis_index("s") * _NC + lax.axis_index("c")
    base = wid * _BPW

    pltpu.sync_copy(center_h.at[pl.ds(base, _BPW)], cidx)
    pltpu.sync_copy(context_h.at[pl.ds(base, _BPW)], xidx)

    def issue_body(g, carry):
        cvec = cidx[pl.ds(g * _L, _L)]
        xvec = xidx[pl.ds(g * _L, _L)]
        for k in range(_L):
            p = g * (_L // 2) + k // 2
            off = (k % 2) * _D
            pltpu.async_copy(in_h.at[cvec[k]],
                             vbuf.at[p, pl.ds(off, _D)], sem_v)
            pltpu.async_copy(oute_h.at[xvec[k]],
                             ubuf.at[p, pl.ds(off, _D)], sem_u)
        return carry

    lax.fori_loop(0, _BPW // _L, issue_body, 0)

    pltpu.make_async_copy(in_h.at[pl.ds(0, _BPW // 2)], vbuf, sem_v).wait()
    pltpu.make_async_copy(oute_h.at[pl.ds(0, _BPW // 2)], ubuf, sem_u).wait()

    iota = lax.iota(jnp.int32, _L)
    last = jnp.full((_L,), _L - 1, jnp.int32)

    def group_body(g, carry):
        outv = jnp.zeros((_L,), jnp.float32)
        for k in range(_L):
            p = g * (_L // 2) + k // 2
            off = (k % 2) * _D
            acc = vbuf[p, pl.ds(off, _L)] * ubuf[p, pl.ds(off, _L)]
            for c in range(1, _D // _L):
                acc = acc + (vbuf[p, pl.ds(off + c * _L, _L)]
                             * ubuf[p, pl.ds(off + c * _L, _L)])
            tot = jnp.cumsum(acc)
            bcast = tot.at[last].get(mode="promise_in_bounds")
            outv = jnp.where(iota == k, bcast, outv)
        obuf[pl.ds(g * _L, _L)] = outv
        return carry

    lax.fori_loop(0, _BPW // _L, group_body, 0)

    pltpu.sync_copy(obuf, o_h.at[pl.ds(base, _BPW)])


def kernel(center, context, in_embed, out_embed):
    return _w2v(center.astype(jnp.int32), context.astype(jnp.int32),
                in_embed, out_embed)


# speedup vs baseline: 1.5791x; 1.0003x over previous
import functools

import jax
import jax.numpy as jnp
from jax import lax
from jax.experimental import pallas as pl
from jax.experimental.pallas import tpu as pltpu
from jax.experimental.pallas import tpu_sc as plsc

_D = 64
_B = 16384
_NC, _NS, _L = 2, 16, 16
_NW = _NC * _NS
_BPW = _B // _NW
_NQ = 8

_mesh = plsc.VectorSubcoreMesh(core_axis_name="c", subcore_axis_name="s")


@functools.partial(
    pl.kernel,
    mesh=_mesh,
    out_type=jax.ShapeDtypeStruct((_B,), jnp.float32),
    compiler_params=pltpu.CompilerParams(
        needs_layout_passes=False, use_tc_tiling_on_sc=True),
    scratch_types=[
        pltpu.VMEM((_BPW,), jnp.int32),
        pltpu.VMEM((_BPW,), jnp.int32),
        pltpu.VMEM((_BPW // 2, 2 * _D), jnp.float32),
        pltpu.VMEM((_BPW // 2, 2 * _D), jnp.float32),
        pltpu.VMEM((_BPW,), jnp.float32),
        [pltpu.SemaphoreType.DMA] * _NQ,
    ],
)
def _w2v(center_h, context_h, in_h, oute_h, o_h, cidx, xidx, vbuf, ubuf,
         obuf, sems):
    wid = lax.axis_index("s") * _NC + lax.axis_index("c")
    base = wid * _BPW

    pltpu.sync_copy(center_h.at[pl.ds(base, _BPW)], cidx)
    pltpu.sync_copy(context_h.at[pl.ds(base, _BPW)], xidx)

    def issue_body(g, carry):
        cvec = cidx[pl.ds(g * _L, _L)]
        xvec = xidx[pl.ds(g * _L, _L)]
        for k in range(_L):
            p = g * (_L // 2) + k // 2
            off = (k % 2) * _D
            pltpu.async_copy(in_h.at[cvec[k]],
                             vbuf.at[p, pl.ds(off, _D)], sems[k % _NQ])
            pltpu.async_copy(oute_h.at[xvec[k]],
                             ubuf.at[p, pl.ds(off, _D)], sems[k % _NQ])
        return carry

    lax.fori_loop(0, _BPW // _L, issue_body, 0)

    for q in range(_NQ):
        pltpu.make_async_copy(
            in_h.at[pl.ds(0, _BPW // _NQ)],
            vbuf.at[pl.ds(0, _BPW // _NQ)],
            sems[q]).wait()

    iota = lax.iota(jnp.int32, _L)
    last = jnp.full((_L,), _L - 1, jnp.int32)

    def group_body(g, carry):
        outv = jnp.zeros((_L,), jnp.float32)
        for k in range(_L):
            p = g * (_L // 2) + k // 2
            off = (k % 2) * _D
            acc = vbuf[p, pl.ds(off, _L)] * ubuf[p, pl.ds(off, _L)]
            for c in range(1, _D // _L):
                acc = acc + (vbuf[p, pl.ds(off + c * _L, _L)]
                             * ubuf[p, pl.ds(off + c * _L, _L)])
            tot = jnp.cumsum(acc)
            bcast = tot.at[last].get(mode="promise_in_bounds")
            outv = jnp.where(iota == k, bcast, outv)
        obuf[pl.ds(g * _L, _L)] = outv
        return carry

    lax.fori_loop(0, _BPW // _L, group_body, 0)

    pltpu.sync_copy(obuf, o_h.at[pl.ds(base, _BPW)])


def kernel(center, context, in_embed, out_embed):
    return _w2v(center.astype(jnp.int32), context.astype(jnp.int32),
                in_embed, out_embed)
